# baseline (speedup 1.0000x reference)
import jax
import jax.numpy as jnp
from jax import lax
from jax.experimental import pallas as pl
from jax.experimental.pallas import tpu as pltpu
from jax.experimental.pallas import tpu_sc as plsc

NN = 10000
EE = 160000
DH = 128
NCORES = 2
NSUB = 16
EDGES_PER_TILE = EE // NSUB
KB = 80
EPT_PAD = 10080
NBATCH = EPT_PAD // KB
ROWS_PER_TILE = 624
ROWS_LAST = NN - 15 * ROWS_PER_TILE
FIN_CHUNK = 16


def _hop(table_ref, dst4_ref, acc, srcflat, valbuf, didx, rows,
         gsem, ssem, dsem, s):
    def issue(b, k):
        pltpu.async_copy(table_ref.at[srcflat.at[pl.ds(b * KB, KB)]],
                         rows[k], gsem[k])
        pltpu.async_copy(dst4_ref.at[s * NBATCH + b], didx[k], dsem[k])

    def wait_gather(b, k):
        pltpu.make_async_copy(table_ref.at[srcflat.at[pl.ds(b * KB, KB)]],
                              rows[k], gsem[k]).wait()

    def wait_didx(b, k):
        pltpu.make_async_copy(dst4_ref.at[s * NBATCH + b], didx[k],
                              dsem[k]).wait()

    def scale(b, k):
        for g in range(KB // 16):
            v16 = valbuf[pl.ds(b * KB + g * 16, 16)]
            for e in range(16):
                r = g * 16 + e
                v = v16[e]
                for q in range(DH // 16):
                    sl = pl.ds(q * 16, 16)
                    rows[k][r, sl] = rows[k][r, sl] * v

    def start_scatter(b, k):
        wait_didx(b, k)
        pltpu.async_copy(rows[k], acc.at[didx[k].at[0]], ssem[k], add=True)

    def wait_scatter(b, k):
        pltpu.make_async_copy(rows[k], acc.at[didx[k].at[0]],
                              ssem[k]).wait()

    issue(0, 0)

    def pair(p, carry):
        b0 = 2 * p

        wait_gather(b0, 0)

        @pl.when(p > 0)
        def _():
            wait_scatter(b0 - 1, 1)

        issue(b0 + 1, 1)
        scale(b0, 0)
        start_scatter(b0, 0)

        wait_gather(b0 + 1, 1)
        scale(b0 + 1, 1)
        wait_scatter(b0, 0)

        @pl.when(p < NBATCH // 2 - 1)
        def _():
            issue(b0 + 2, 0)

        start_scatter(b0 + 1, 1)
        return carry

    lax.fori_loop(0, NBATCH // 2, pair, 0)
    wait_scatter(NBATCH - 1, 1)


def _body(xs_ref, src2_ref, dst4_ref, vals2_ref, alpha_ref, zeros_ref,
          out_ref, ax_ref,
          acc, srcflat, valbuf, didx0, didx1, rows0, rows1,
          avec, gsem0, gsem1, ssem0, ssem1, dsem0, dsem1):
    rows = (rows0, rows1)
    didx = (didx0, didx1)
    gsem = (gsem0, gsem1)
    ssem = (ssem0, ssem1)
    dsem = (dsem0, dsem1)
    c = lax.axis_index("c")
    s = lax.axis_index("s")
    coff = c * NN
    stripe = s * ROWS_PER_TILE
    is_last = s == NSUB - 1

    def _zero_acc():
        pltpu.sync_copy(zeros_ref.at[pl.ds(0, ROWS_PER_TILE)],
                        acc.at[pl.ds(stripe, ROWS_PER_TILE)])

        @pl.when(is_last)
        def _():
            pltpu.sync_copy(
                zeros_ref.at[pl.ds(0, ROWS_LAST - ROWS_PER_TILE)],
                acc.at[pl.ds(stripe + ROWS_PER_TILE,
                             ROWS_LAST - ROWS_PER_TILE)])

    with jax.named_scope("stage_in"):
        pltpu.sync_copy(alpha_ref, avec)
        pltpu.sync_copy(src2_ref.at[c, s], srcflat)
        pltpu.sync_copy(vals2_ref.at[s], valbuf)
        _zero_acc()
        plsc.subcore_barrier()

    with jax.named_scope("hop1"):
        _hop(xs_ref, dst4_ref, acc, srcflat, valbuf, didx, rows,
             gsem, ssem, dsem, s)
        plsc.subcore_barrier()

    with jax.named_scope("ax_out"):
        pltpu.sync_copy(acc.at[pl.ds(stripe, ROWS_PER_TILE)],
                        ax_ref.at[pl.ds(coff + stripe, ROWS_PER_TILE)])

        @pl.when(is_last)
        def _():
            pltpu.sync_copy(
                acc.at[pl.ds(stripe + ROWS_PER_TILE,
                             ROWS_LAST - ROWS_PER_TILE)],
                ax_ref.at[pl.ds(coff + stripe + ROWS_PER_TILE,
                                ROWS_LAST - ROWS_PER_TILE)])

        _zero_acc()
        plsc.subcore_barrier()

    with jax.named_scope("hop2"):
        _hop(ax_ref, dst4_ref, acc, srcflat, valbuf, didx, rows,
             gsem, ssem, dsem, s)
        plsc.subcore_barrier()

    a = avec[...]
    alph = 1.0 / (1.0 + jnp.exp(-a))

    def fin_chunk(rbase, nrows):
        cp_a = pltpu.async_copy(acc.at[pl.ds(rbase, nrows)],
                                rows0.at[pl.ds(0, nrows)], gsem0)
        cp_x = pltpu.async_copy(xs_ref.at[pl.ds(coff + rbase, nrows)],
                                rows1.at[pl.ds(0, nrows)], gsem1)
        cp_a.wait()
        cp_x.wait()
        for r in range(nrows):
            for q in range(DH // 16):
                sl = pl.ds(q * 16, 16)
                rows0[r, sl] = alph * rows0[r, sl] - rows1[r, sl]
        pltpu.sync_copy(rows0.at[pl.ds(0, nrows)],
                        out_ref.at[pl.ds(coff + rbase, nrows)])

    def fin16(k, carry):
        fin_chunk(pl.multiple_of(stripe + k * FIN_CHUNK, FIN_CHUNK),
                  FIN_CHUNK)
        return carry

    with jax.named_scope("finalize"):
        nfin = jnp.where(is_last, ROWS_LAST // FIN_CHUNK,
                         ROWS_PER_TILE // FIN_CHUNK)
        lax.fori_loop(0, nfin, fin16, 0)


@jax.jit
def _diffuse(xs, src2, dst4, vals2, alpha16, zeros):
    mesh = plsc.VectorSubcoreMesh(core_axis_name="c", subcore_axis_name="s")
    f = pl.kernel(
        _body,
        mesh=mesh,
        out_type=[
            jax.ShapeDtypeStruct((NCORES * NN, DH), jnp.float32),
            jax.ShapeDtypeStruct((NCORES * NN, DH), jnp.float32),
        ],
        scratch_types=[
            pltpu.VMEM_SHARED((NN, DH), jnp.float32),
            pltpu.VMEM((EPT_PAD,), jnp.int32),
            pltpu.VMEM((EPT_PAD,), jnp.float32),
            pltpu.VMEM((1, KB), jnp.int32),
            pltpu.VMEM((1, KB), jnp.int32),
            pltpu.VMEM((KB, DH), jnp.float32),
            pltpu.VMEM((KB, DH), jnp.float32),
            pltpu.VMEM((16,), jnp.float32),
        ] + [pltpu.SemaphoreType.DMA] * 6,
    )
    return f(xs, src2, dst4, vals2, alpha16, zeros)


def kernel(t, x, adj_indices, adj_values, alpha_train):
    del t
    n, d = x.shape
    xs = x.reshape(n, NCORES, DH).transpose(1, 0, 2).reshape(NCORES * n, DH)
    src = adj_indices[0].reshape(NSUB, EDGES_PER_TILE)
    dst = adj_indices[1].reshape(NSUB, EDGES_PER_TILE)
    vals = adj_values.reshape(NSUB, EDGES_PER_TILE)
    pad_i = jnp.zeros((NSUB, EPT_PAD - EDGES_PER_TILE), jnp.int32)
    pad_f = jnp.zeros((NSUB, EPT_PAD - EDGES_PER_TILE), jnp.float32)
    srcp = jnp.concatenate([src, pad_i], axis=1)
    dstp = jnp.concatenate([dst, pad_i], axis=1)
    vals2 = jnp.concatenate([vals, pad_f], axis=1)
    src2 = jnp.stack([srcp, srcp + n])
    dst4 = dstp.reshape(NSUB * NBATCH, 1, KB)
    alpha16 = jnp.broadcast_to(alpha_train.astype(jnp.float32), (16,))
    zeros = jnp.zeros((ROWS_PER_TILE, DH), jnp.float32)
    out, _ = _diffuse(xs, src2, dst4, vals2, alpha16, zeros)
    return out.reshape(NCORES, n, DH).transpose(1, 0, 2).reshape(n, d)


# speedup vs baseline: 1.2164x; 1.2164x over previous
import jax
import jax.numpy as jnp
from jax import lax
from jax.experimental import pallas as pl
from jax.experimental.pallas import tpu as pltpu
from jax.experimental.pallas import tpu_sc as plsc

NN = 10000
EE = 160000
DH = 128
NCORES = 2
NSUB = 16
EDGES_PER_TILE = EE // NSUB
KB = 80
NBATCH = EDGES_PER_TILE // KB
ROWS_PER_TILE = 624
ROWS_LAST = NN - 15 * ROWS_PER_TILE
FIN_CHUNK = 16
NFIN = ROWS_PER_TILE // FIN_CHUNK


def _hop(table_ref, dst3_ref, acc, srcflat, valbuf, didx, rows,
         gsem, ssem, dsem, s):
    def issue(b, k):
        pltpu.async_copy(table_ref.at[srcflat.at[pl.ds(b * KB, KB)]],
                         rows[k], gsem[k])
        pltpu.async_copy(dst3_ref.at[s * NBATCH + b], didx[k], dsem[k])

    def wait_gather(b, k):
        pltpu.make_async_copy(table_ref.at[srcflat.at[pl.ds(b * KB, KB)]],
                              rows[k], gsem[k]).wait()

    def wait_didx(b, k):
        pltpu.make_async_copy(dst3_ref.at[s * NBATCH + b], didx[k],
                              dsem[k]).wait()

    def scale(b, k):
        for g in range(KB // 16):
            v16 = valbuf[pl.ds(b * KB + g * 16, 16)]
            for e in range(16):
                r = g * 16 + e
                v = v16[e]
                for q in range(DH // 16):
                    sl = pl.ds(q * 16, 16)
                    rows[k][r, sl] = rows[k][r, sl] * v

    def start_scatter(b, k):
        wait_didx(b, k)
        pltpu.async_copy(rows[k], acc.at[didx[k].at[0]], ssem[k], add=True)

    def wait_scatter(b, k):
        pltpu.make_async_copy(rows[k], acc.at[didx[k].at[0]],
                              ssem[k]).wait()

    issue(0, 0)

    def pair(p, carry):
        b0 = 2 * p

        wait_gather(b0, 0)

        @pl.when(p > 0)
        def _():
            wait_scatter(b0 - 1, 1)

        issue(b0 + 1, 1)
        scale(b0, 0)
        start_scatter(b0, 0)

        wait_gather(b0 + 1, 1)
        scale(b0 + 1, 1)
        wait_scatter(b0, 0)
        issue(b0 + 2, 0)
        start_scatter(b0 + 1, 1)
        return carry

    npair = (NBATCH - 1) // 2
    lax.fori_loop(0, npair, pair, 0)

    blast = NBATCH - 1
    wait_gather(blast, 0)
    wait_scatter(blast - 1, 1)
    scale(blast, 0)
    start_scatter(blast, 0)
    wait_scatter(blast, 0)


def _body(xs_ref, src2_ref, dst3_ref, vals2_ref, alpha_ref, zeros_ref,
          out_ref, ax_ref,
          acc, srcflat, valbuf, didx0, didx1, rows0, rows1,
          avec, gsem0, gsem1, ssem0, ssem1, dsem0, dsem1):
    rows = (rows0, rows1)
    didx = (didx0, didx1)
    gsem = (gsem0, gsem1)
    ssem = (ssem0, ssem1)
    dsem = (dsem0, dsem1)
    c = lax.axis_index("c")
    s = lax.axis_index("s")
    coff = c * NN
    stripe = s * ROWS_PER_TILE
    is_last = s == NSUB - 1

    def _zero_acc():
        pltpu.sync_copy(zeros_ref.at[pl.ds(0, ROWS_PER_TILE)],
                        acc.at[pl.ds(stripe, ROWS_PER_TILE)])

        @pl.when(is_last)
        def _():
            pltpu.sync_copy(
                zeros_ref.at[pl.ds(0, ROWS_LAST - ROWS_PER_TILE)],
                acc.at[pl.ds(stripe + ROWS_PER_TILE,
                             ROWS_LAST - ROWS_PER_TILE)])

    with jax.named_scope("stage_in"):
        pltpu.sync_copy(alpha_ref, avec)
        pltpu.sync_copy(src2_ref.at[c, s], srcflat)
        pltpu.sync_copy(vals2_ref.at[s], valbuf)
        _zero_acc()
        plsc.subcore_barrier()

    with jax.named_scope("hop1"):
        _hop(xs_ref, dst3_ref, acc, srcflat, valbuf, didx, rows,
             gsem, ssem, dsem, s)
        plsc.subcore_barrier()

    with jax.named_scope("ax_out"):
        pltpu.sync_copy(acc.at[pl.ds(stripe, ROWS_PER_TILE)],
                        ax_ref.at[pl.ds(coff + stripe, ROWS_PER_TILE)])

        @pl.when(is_last)
        def _():
            pltpu.sync_copy(
                acc.at[pl.ds(stripe + ROWS_PER_TILE,
                             ROWS_LAST - ROWS_PER_TILE)],
                ax_ref.at[pl.ds(coff + stripe + ROWS_PER_TILE,
                                ROWS_LAST - ROWS_PER_TILE)])

        _zero_acc()
        plsc.subcore_barrier()

    with jax.named_scope("hop2"):
        _hop(ax_ref, dst3_ref, acc, srcflat, valbuf, didx, rows,
             gsem, ssem, dsem, s)
        plsc.subcore_barrier()

    a = avec[...]
    alph = 1.0 / (1.0 + jnp.exp(-a))

    def fin_chunk(rbase, nrows):
        cp_a = pltpu.async_copy(acc.at[pl.ds(rbase, nrows)],
                                rows0.at[pl.ds(0, nrows)], gsem0)
        cp_x = pltpu.async_copy(xs_ref.at[pl.ds(coff + rbase, nrows)],
                                rows1.at[pl.ds(0, nrows)], gsem1)
        cp_a.wait()
        cp_x.wait()
        for r in range(nrows):
            for q in range(DH // 16):
                sl = pl.ds(q * 16, 16)
                rows0[r, sl] = alph * rows0[r, sl] - rows1[r, sl]
        pltpu.sync_copy(rows0.at[pl.ds(0, nrows)],
                        out_ref.at[pl.ds(coff + rbase, nrows)])

    def fin48(k, carry):
        fin_chunk(pl.multiple_of(stripe + k * 48, FIN_CHUNK), 48)
        return carry

    with jax.named_scope("finalize"):
        lax.fori_loop(0, ROWS_PER_TILE // 48, fin48, 0)

        @pl.when(is_last)
        def _():
            fin_chunk(stripe + ROWS_PER_TILE, ROWS_LAST - ROWS_PER_TILE)


@jax.jit
def _diffuse(xs, src2, dst3, vals2, alpha16, zeros):
    mesh = plsc.VectorSubcoreMesh(core_axis_name="c", subcore_axis_name="s")
    f = pl.kernel(
        _body,
        mesh=mesh,
        out_type=[
            jax.ShapeDtypeStruct((NCORES * NN, DH), jnp.float32),
            jax.ShapeDtypeStruct((NCORES * NN, DH), jnp.float32),
        ],
        scratch_types=[
            pltpu.VMEM_SHARED((NN, DH), jnp.float32),
            pltpu.VMEM((EDGES_PER_TILE,), jnp.int32),
            pltpu.VMEM((EDGES_PER_TILE,), jnp.float32),
            pltpu.VMEM((1, KB), jnp.int32),
            pltpu.VMEM((1, KB), jnp.int32),
            pltpu.VMEM((KB, DH), jnp.float32),
            pltpu.VMEM((KB, DH), jnp.float32),
            pltpu.VMEM((16,), jnp.float32),
            pltpu.SemaphoreType.DMA,
            pltpu.SemaphoreType.DMA,
            pltpu.SemaphoreType.DMA,
            pltpu.SemaphoreType.DMA,
            pltpu.SemaphoreType.DMA,
            pltpu.SemaphoreType.DMA,
        ],
    )
    return f(xs, src2, dst3, vals2, alpha16, zeros)


def kernel(t, x, adj_indices, adj_values, alpha_train):
    del t
    n, d = x.shape
    xs = x.reshape(n, NCORES, DH).transpose(1, 0, 2).reshape(NCORES * n, DH)
    src = adj_indices[0]
    dst = adj_indices[1]
    src2 = jnp.stack([src, src + n]).reshape(NCORES, NSUB, EDGES_PER_TILE)
    dst3 = dst.reshape(NSUB * NBATCH, 1, KB)
    vals2 = adj_values.reshape(NSUB, EDGES_PER_TILE)
    alpha16 = jnp.broadcast_to(alpha_train.astype(jnp.float32), (16,))
    zeros = jnp.zeros((ROWS_PER_TILE, DH), jnp.float32)
    out, _ = _diffuse(xs, src2, dst3, vals2, alpha16, zeros)
    return out.reshape(NCORES, n, DH).transpose(1, 0, 2).reshape(n, d)
